# Initial kernel scaffold; baseline (speedup 1.0000x reference)
import jax
import jax.numpy as jnp
from jax.experimental import pallas as pl


def kernel(x, edge_index, W0, W1, W_out):
    raise NotImplementedError("write your pallas kernel here")


# speedup vs baseline: 4.9198x; 4.9198x over previous
import functools

import jax
import jax.numpy as jnp
from jax import lax
from jax.experimental import pallas as pl
from jax.experimental.pallas import tpu as pltpu
from jax.experimental.pallas import tpu_sc as plsc

N = 10000
D = 128
DH = 64
E = 320000
NW = 32
NCH = 80
C = 128
EPAD = NW * NCH * C
NP = 10240
RPT = NP // 16
DEGW = 16

_mesh = plsc.VectorSubcoreMesh(core_axis_name="c", subcore_axis_name="s")


def _sc_degree_body(row_hbm, ones_hbm, zeros_hbm, out_hbm, idx_v, ones_v,
                    zbuf_v, acc_sh):
    c = lax.axis_index("c")
    s = lax.axis_index("s")
    wid = c * 16 + s
    sl = pl.ds(s * RPT, RPT)
    pltpu.sync_copy(zeros_hbm, zbuf_v)
    pltpu.sync_copy(zbuf_v, acc_sh.at[sl])
    pltpu.sync_copy(row_hbm.at[wid], idx_v)
    pltpu.sync_copy(ones_hbm, ones_v)
    plsc.subcore_barrier()

    def body(j, carry):
        pltpu.sync_copy(ones_v, acc_sh.at[idx_v.at[j]], add=True)
        return carry

    lax.fori_loop(0, NCH, body, 0)
    plsc.subcore_barrier()
    pltpu.sync_copy(acc_sh.at[sl], zbuf_v)
    pltpu.sync_copy(zbuf_v, out_hbm.at[c, sl])


def _sc_aggregate_body(ya_hbm, yb_hbm, row_hbm, col_hbm, zeros_hbm, out_hbm,
                       row_v, col_v, buf_v, zbuf_v, acc_sh):
    c = lax.axis_index("c")
    s = lax.axis_index("s")
    wid = c * 16 + s
    sl = pl.ds(s * RPT, RPT)
    pltpu.sync_copy(row_hbm.at[wid], row_v)
    pltpu.sync_copy(col_hbm.at[wid], col_v)
    for dh, y_hbm in enumerate((ya_hbm, yb_hbm)):
        pltpu.sync_copy(zeros_hbm, zbuf_v)
        pltpu.sync_copy(zbuf_v, acc_sh.at[sl])
        plsc.subcore_barrier()

        def body(j, carry):
            pltpu.sync_copy(y_hbm.at[col_v.at[j]], buf_v)
            pltpu.sync_copy(buf_v, acc_sh.at[row_v.at[j]], add=True)
            return carry

        lax.fori_loop(0, NCH, body, 0)
        plsc.subcore_barrier()
        pltpu.sync_copy(acc_sh.at[sl], zbuf_v)
        pltpu.sync_copy(zbuf_v, out_hbm.at[c, dh, sl])
        plsc.subcore_barrier()


def _build_sc_degree(interpret=False, mesh=None):
    return functools.partial(
        pl.kernel,
        out_type=jax.ShapeDtypeStruct((2, NP, DEGW), jnp.float32),
        mesh=mesh or _mesh,
        compiler_params=pltpu.CompilerParams(use_tc_tiling_on_sc=False),
        scratch_types=[
            pltpu.VMEM((NCH, C), jnp.int32),
            pltpu.VMEM((C, DEGW), jnp.float32),
            pltpu.VMEM((RPT, DEGW), jnp.float32),
            pltpu.VMEM_SHARED((NP, DEGW), jnp.float32),
        ],
        interpret=interpret,
    )(_sc_degree_body)


def _build_sc_aggregate(interpret=False, mesh=None):
    return functools.partial(
        pl.kernel,
        out_type=jax.ShapeDtypeStruct((2, 2, NP, DH), jnp.float32),
        mesh=mesh or _mesh,
        compiler_params=pltpu.CompilerParams(use_tc_tiling_on_sc=False),
        scratch_types=[
            pltpu.VMEM((NCH, C), jnp.int32),
            pltpu.VMEM((NCH, C), jnp.int32),
            pltpu.VMEM((C, DH), jnp.float32),
            pltpu.VMEM((RPT, DH), jnp.float32),
            pltpu.VMEM_SHARED((NP, DH), jnp.float32),
        ],
        interpret=interpret,
    )(_sc_aggregate_body)


_sc_degree = _build_sc_degree()
_sc_aggregate = _build_sc_aggregate()


def _dinv_from_parts(degp):
    deg = jnp.sum(degp, axis=(0, 2))[:N]
    return 1.0 / jnp.sqrt(deg + 1e-12)


def _assemble(parts):
    return jnp.concatenate(
        [parts[0, 0, :N, :] + parts[1, 0, :N, :],
         parts[0, 1, :N, :] + parts[1, 1, :N, :]], axis=1)


def _tc_first_body(x_ref, w_ref, degp_ref, oa_ref, ob_ref):
    dinv = _dinv_from_parts(degp_ref[...])
    xw = lax.dot_general(x_ref[...], w_ref[...], (((1,), (1,)), ((), ())),
                         preferred_element_type=jnp.float32)
    y = xw * dinv[:, None]
    oa_ref[...] = y[:, :DH]
    ob_ref[...] = y[:, DH:]


def _tc_mid_body(parts_ref, w_ref, degp_ref, oa_ref, ob_ref):
    dinv = _dinv_from_parts(degp_ref[...])
    agg = _assemble(parts_ref[...])
    h = jnp.maximum(agg * dinv[:, None], 0.0)
    hw = lax.dot_general(h, w_ref[...], (((1,), (1,)), ((), ())),
                         preferred_element_type=jnp.float32)
    y = hw * dinv[:, None]
    oa_ref[...] = y[:, :DH]
    ob_ref[...] = y[:, DH:]


def _tc_last_body(parts_ref, degp_ref, o_ref):
    dinv = _dinv_from_parts(degp_ref[...])
    o_ref[...] = _assemble(parts_ref[...]) * dinv[:, None]


_out_halves = [jax.ShapeDtypeStruct((N, DH), jnp.float32)] * 2
_tc_first = pl.pallas_call(_tc_first_body, out_shape=_out_halves)
_tc_mid = pl.pallas_call(_tc_mid_body, out_shape=_out_halves)
_tc_last = pl.pallas_call(
    _tc_last_body, out_shape=jax.ShapeDtypeStruct((N, D), jnp.float32))


def _prep_edges(edge_index):
    ei = edge_index.astype(jnp.int32)
    pad = EPAD - E
    row_p = jnp.concatenate(
        [ei[0], jnp.full((pad,), N, jnp.int32)]).reshape(NW, NCH, C)
    col_p = jnp.concatenate(
        [ei[1], jnp.zeros((pad,), jnp.int32)]).reshape(NW, NCH, C)
    return row_p, col_p


def kernel(x, edge_index, W0, W1, W_out):
    row_p, col_p = _prep_edges(edge_index)
    ones16 = jnp.zeros((C, DEGW), jnp.float32).at[:, 0].set(1.0)
    zeros16 = jnp.zeros((RPT, DEGW), jnp.float32)
    zeros64 = jnp.zeros((RPT, DH), jnp.float32)

    degp = _sc_degree(row_p, ones16, zeros16)
    ya, yb = _tc_first(x, W0, degp)
    p0 = _sc_aggregate(ya, yb, row_p, col_p, zeros64)
    ya, yb = _tc_mid(p0, W1, degp)
    p1 = _sc_aggregate(ya, yb, row_p, col_p, zeros64)
    ya, yb = _tc_mid(p1, W_out, degp)
    p2 = _sc_aggregate(ya, yb, row_p, col_p, zeros64)
    return _tc_last(p2, degp)


# speedup vs baseline: 5.5361x; 1.1253x over previous
import functools

import jax
import jax.numpy as jnp
from jax import lax
from jax.experimental import pallas as pl
from jax.experimental.pallas import tpu as pltpu
from jax.experimental.pallas import tpu_sc as plsc

N = 10000
D = 128
DH = 64
E = 320000
NW = 32
NCH = 80
C = 128
EPAD = NW * NCH * C
NP = 10240
RPT = NP // 16
DEGW = 16

_mesh = plsc.VectorSubcoreMesh(core_axis_name="c", subcore_axis_name="s")


def _sc_degree_body(row_hbm, ones_hbm, zeros_hbm, out_hbm, idx_v, ones_v,
                    zbuf_v, dsem, acc_sh):
    c = lax.axis_index("c")
    s = lax.axis_index("s")
    wid = c * 16 + s
    sl = pl.ds(s * RPT, RPT)
    pltpu.sync_copy(zeros_hbm, zbuf_v)
    pltpu.sync_copy(zbuf_v, acc_sh.at[sl])
    pltpu.sync_copy(row_hbm.at[wid], idx_v)
    pltpu.sync_copy(ones_hbm, ones_v)
    plsc.subcore_barrier()

    def body(j, carry):
        pltpu.sync_copy(ones_v, acc_sh.at[idx_v.at[j]], add=True)
        return carry

    lax.fori_loop(0, NCH, body, 0)
    plsc.subcore_barrier()
    pltpu.sync_copy(acc_sh.at[sl], zbuf_v)
    pltpu.sync_copy(zbuf_v, out_hbm.at[c, sl])


NBUF = 4
ZCH = RPT // 4


def _sc_aggregate_body(ya_hbm, yb_hbm, row_hbm, col_hbm, zeros_hbm, out_hbm,
                       row_v, col_v, zbuf_v,
                       b0, b1, b2, b3, g0, g1, g2, g3, s0, s1, s2, s3,
                       acc_sh):
    bufs = (b0, b1, b2, b3)
    gsems = (g0, g1, g2, g3)
    ssems = (s0, s1, s2, s3)
    c = lax.axis_index("c")
    s = lax.axis_index("s")
    wid = c * 16 + s
    pltpu.sync_copy(row_hbm.at[wid], row_v)
    pltpu.sync_copy(col_hbm.at[wid], col_v)
    for dh, y_hbm in enumerate((ya_hbm, yb_hbm)):
        pltpu.sync_copy(zeros_hbm, zbuf_v)
        for t in range(4):
            pltpu.sync_copy(zbuf_v, acc_sh.at[pl.ds(s * RPT + t * ZCH, ZCH)])
        plsc.subcore_barrier()

        for b in range(NBUF):
            pltpu.async_copy(y_hbm.at[col_v.at[b]], bufs[b], gsems[b])

        def outer(o, carry):
            for b in range(NBUF):
                j = o * NBUF + b
                pltpu.make_async_copy(
                    y_hbm.at[col_v.at[j]], bufs[b], gsems[b]).wait()
                pltpu.async_copy(
                    bufs[b], acc_sh.at[row_v.at[j]], ssems[b], add=True)

                @pl.when(o < (NCH // NBUF) - 1)
                def _():
                    pltpu.make_async_copy(
                        bufs[b], acc_sh.at[row_v.at[j]], ssems[b]).wait()
                    pltpu.async_copy(
                        y_hbm.at[col_v.at[j + NBUF]], bufs[b], gsems[b])
            return carry

        lax.fori_loop(0, NCH // NBUF, outer, 0)
        for b in range(NBUF):
            pltpu.make_async_copy(
                bufs[b], acc_sh.at[row_v.at[NCH - NBUF + b]], ssems[b]).wait()
        plsc.subcore_barrier()
        for t in range(4):
            sl2 = pl.ds(s * RPT + t * ZCH, ZCH)
            pltpu.sync_copy(acc_sh.at[sl2], zbuf_v)
            pltpu.sync_copy(zbuf_v, out_hbm.at[c, dh, sl2])
        plsc.subcore_barrier()


def _build_sc_degree(interpret=False, mesh=None):
    return functools.partial(
        pl.kernel,
        out_type=jax.ShapeDtypeStruct((2, NP, DEGW), jnp.float32),
        mesh=mesh or _mesh,
        compiler_params=pltpu.CompilerParams(use_tc_tiling_on_sc=False),
        scratch_types=[
            pltpu.VMEM((NCH, C), jnp.int32),
            pltpu.VMEM((C, DEGW), jnp.float32),
            pltpu.VMEM((RPT, DEGW), jnp.float32),
            pltpu.SemaphoreType.DMA,
            pltpu.VMEM_SHARED((NP, DEGW), jnp.float32),
        ],
        interpret=interpret,
    )(_sc_degree_body)


def _build_sc_aggregate(interpret=False, mesh=None):
    return functools.partial(
        pl.kernel,
        out_type=jax.ShapeDtypeStruct((2, 2, NP, DH), jnp.float32),
        mesh=mesh or _mesh,
        compiler_params=pltpu.CompilerParams(use_tc_tiling_on_sc=False),
        scratch_types=(
            [pltpu.VMEM((NCH, C), jnp.int32)] * 2
            + [pltpu.VMEM((ZCH, DH), jnp.float32)]
            + [pltpu.VMEM((C, DH), jnp.float32)] * NBUF
            + [pltpu.SemaphoreType.DMA] * (2 * NBUF)
            + [pltpu.VMEM_SHARED((NP, DH), jnp.float32)]
        ),
        interpret=interpret,
    )(_sc_aggregate_body)


_sc_degree = _build_sc_degree()
_sc_aggregate = _build_sc_aggregate()


def _dinv_from_parts(degp):
    deg = jnp.sum(degp, axis=(0, 2))[:N]
    return 1.0 / jnp.sqrt(deg + 1e-12)


def _assemble(parts):
    return jnp.concatenate(
        [parts[0, 0, :N, :] + parts[1, 0, :N, :],
         parts[0, 1, :N, :] + parts[1, 1, :N, :]], axis=1)


def _tc_first_body(x_ref, w_ref, degp_ref, oa_ref, ob_ref):
    dinv = _dinv_from_parts(degp_ref[...])
    xw = lax.dot_general(x_ref[...], w_ref[...], (((1,), (1,)), ((), ())),
                         preferred_element_type=jnp.float32)
    y = xw * dinv[:, None]
    oa_ref[...] = y[:, :DH]
    ob_ref[...] = y[:, DH:]


def _tc_mid_body(parts_ref, w_ref, degp_ref, oa_ref, ob_ref):
    dinv = _dinv_from_parts(degp_ref[...])
    agg = _assemble(parts_ref[...])
    h = jnp.maximum(agg * dinv[:, None], 0.0)
    hw = lax.dot_general(h, w_ref[...], (((1,), (1,)), ((), ())),
                         preferred_element_type=jnp.float32)
    y = hw * dinv[:, None]
    oa_ref[...] = y[:, :DH]
    ob_ref[...] = y[:, DH:]


def _tc_last_body(parts_ref, degp_ref, o_ref):
    dinv = _dinv_from_parts(degp_ref[...])
    o_ref[...] = _assemble(parts_ref[...]) * dinv[:, None]


_out_halves = [jax.ShapeDtypeStruct((N, DH), jnp.float32)] * 2
_tc_first = pl.pallas_call(_tc_first_body, out_shape=_out_halves)
_tc_mid = pl.pallas_call(_tc_mid_body, out_shape=_out_halves)
_tc_last = pl.pallas_call(
    _tc_last_body, out_shape=jax.ShapeDtypeStruct((N, D), jnp.float32))


def _prep_edges(edge_index):
    ei = edge_index.astype(jnp.int32)
    pad = EPAD - E
    row_p = jnp.concatenate(
        [ei[0], jnp.full((pad,), N, jnp.int32)]).reshape(NW, NCH, C)
    col_p = jnp.concatenate(
        [ei[1], jnp.zeros((pad,), jnp.int32)]).reshape(NW, NCH, C)
    return row_p, col_p


def kernel(x, edge_index, W0, W1, W_out):
    row_p, col_p = _prep_edges(edge_index)
    ones16 = jnp.zeros((C, DEGW), jnp.float32).at[:, 0].set(1.0)
    zeros16 = jnp.zeros((RPT, DEGW), jnp.float32)
    zeros64 = jnp.zeros((ZCH, DH), jnp.float32)

    degp = _sc_degree(row_p, ones16, zeros16)
    ya, yb = _tc_first(x, W0, degp)
    p0 = _sc_aggregate(ya, yb, row_p, col_p, zeros64)
    ya, yb = _tc_mid(p0, W1, degp)
    p1 = _sc_aggregate(ya, yb, row_p, col_p, zeros64)
    ya, yb = _tc_mid(p1, W_out, degp)
    p2 = _sc_aggregate(ya, yb, row_p, col_p, zeros64)
    return _tc_last(p2, degp)
